# baseline (device time: 55627 ns/iter reference)
import functools

import jax
import jax.numpy as jnp
from jax import lax
from jax.experimental import pallas as pl
from jax.experimental.pallas import tpu as pltpu

M, N = 1024, 1024
HALF_M = M // 2
HALF_N = N // 2
N_PHASES = 6

PHASES_A = [
    ("pair", "x", False, 512, 0, 0),
    ("a2a", "y", False, 256, 256, 1),
    ("a2a", "z", False, 64, 448, 4),
    ("a2a", "z", True, 16, None, 7),
    ("a2a", "y", True, 64, None, 10),
    ("pair", "x", True, 256, None, 13),
]
PHASES_B = [
    ("a2a", "y", False, 512, 512, 0),
    ("a2a", "z", False, 128, 896, 3),
    ("pair", "x", False, 32, 992, 6),
    ("pair", "x", True, 16, None, 7),
    ("a2a", "z", True, 32, None, 8),
    ("a2a", "y", True, 128, None, 11),
]
N_SEMS = 14
FLOWS = [
    (PHASES_A, 0, 0, 0),
    (PHASES_B, HALF_M, 0, 0),
    (PHASES_A, 0, HALF_N, 1),
    (PHASES_B, HALF_M, HALF_N, 1),
]
N_FLOWS = len(FLOWS)
MAX_LAG = max(f[3] for f in FLOWS)


def kernel(x):
    def body(x_ref, out_ref, accum, recv, send_sems, recv_sems):
        ix = lax.axis_index("x")
        iy = lax.axis_index("y")
        iz = lax.axis_index("z")
        me = (ix, iy, iz)
        dims = {
            "x": (ix, lambda q: (q, iy, iz)),
            "y": (iy, lambda q: (ix, q, iz)),
            "z": (iz, lambda q: (ix, iy, q)),
        }

        def line_peers(dim):
            i_d, coords = dims[dim]
            out = []
            for j in range(3):
                q = j + (j >= i_d).astype(jnp.int32)
                out.append((j, q, coords(q)))
            return out

        barrier_sem = pltpu.get_barrier_semaphore()
        pl.semaphore_signal(
            barrier_sem, inc=1, device_id=(1 - ix, iy, iz),
            device_id_type=pl.DeviceIdType.MESH,
        )
        for dim in ("y", "z"):
            for _, _, dev in line_peers(dim):
                pl.semaphore_signal(
                    barrier_sem, inc=1, device_id=dev,
                    device_id_type=pl.DeviceIdType.MESH,
                )
        pl.semaphore_wait(barrier_sem, 7)

        accum[...] = x_ref[0, 0, 0].astype(jnp.bfloat16)

        def dsr(off, size):
            return pl.ds(pl.multiple_of(off, 16), size)

        pending_sends = []

        def mk(src, dst, send_idx, recv_idx, fi, dev):
            return pltpu.make_async_remote_copy(
                src_ref=src, dst_ref=dst,
                send_sem=send_sems.at[fi, send_idx],
                recv_sem=recv_sems.at[fi, recv_idx],
                device_id=dev, device_id_type=pl.DeviceIdType.MESH,
            )

        def start(fi, ph, off):
            phases, _, c0, _ = FLOWS[fi]
            kind, dim, is_ag, size, rstart, base = phases[ph]
            i_d, coords = dims[dim]
            cols = pl.ds(c0, HALF_N)
            if kind == "pair":
                b = i_d
                if not is_ag:
                    half = size // 2
                    src = accum.at[dsr(off + (1 - b) * half, half), cols]
                    dst = recv.at[dsr(rstart, half), cols]
                else:
                    src = accum.at[dsr(off, size), cols]
                    dst = accum.at[dsr(off, size), cols]
                r = mk(src, dst, base, base, fi, coords(1 - b))
                r.start()
                pending_sends.append(r)
                return
            for j, q, dev in line_peers(dim):
                slot = i_d - (i_d > q).astype(jnp.int32)
                if not is_ag:
                    quarter = size // 4
                    src = accum.at[dsr(off + q * quarter, quarter), cols]
                    dst = recv.at[dsr(rstart + slot * quarter, quarter), cols]
                else:
                    src = accum.at[dsr(off, size), cols]
                    dst = accum.at[dsr(off, size), cols]
                r = mk(src, dst, base + j, base + slot, fi, dev)
                r.start()
                pending_sends.append(r)

        def wait_and_advance(fi, ph, off):
            phases, _, c0, _ = FLOWS[fi]
            kind, dim, is_ag, size, rstart, base = phases[ph]
            i_d, coords = dims[dim]
            cols = pl.ds(c0, HALF_N)

            def wait_recv(dst, sem_idx):
                mk(dst, dst, sem_idx, sem_idx, fi, me).wait_recv()

            if kind == "pair":
                b = i_d
                if not is_ag:
                    half = size // 2
                    keep = off + b * half
                    wait_recv(recv.at[dsr(rstart, half), cols], base)
                    accum[dsr(keep, half), cols] += recv[dsr(rstart, half), cols]
                    return keep
                wait_recv(accum.at[dsr(off + (1 - 2 * b) * size, size), cols], base)
                return off - b * size
            if not is_ag:
                quarter = size // 4
                keep = off + i_d * quarter
                for j in range(3):
                    wait_recv(
                        recv.at[dsr(rstart + j * quarter, quarter), cols], base + j
                    )
                accum[dsr(keep, quarter), cols] += (
                    recv[dsr(rstart, quarter), cols]
                    + recv[dsr(rstart + quarter, quarter), cols]
                    + recv[dsr(rstart + 2 * quarter, quarter), cols]
                )
                return keep
            new_off = off - i_d * size
            for j, q, _ in line_peers(dim):
                wait_recv(accum.at[dsr(new_off + q * size, size), cols], base + j)
            return new_off

        offs = [jnp.int32(FLOWS[fi][1]) for fi in range(N_FLOWS)]
        for fi in range(N_FLOWS):
            if FLOWS[fi][3] == 0:
                start(fi, 0, offs[fi])
        for tau in range(N_PHASES + MAX_LAG):
            for fi in range(N_FLOWS):
                ph = tau - FLOWS[fi][3]
                if ph < 0 or ph >= N_PHASES:
                    continue
                offs[fi] = wait_and_advance(fi, ph, offs[fi])
                if ph + 1 < N_PHASES:
                    start(fi, ph + 1, offs[fi])
            for fi in range(N_FLOWS):
                if FLOWS[fi][3] == tau + 1:
                    start(fi, 0, offs[fi])

        out_ref[...] = accum[...].astype(jnp.float32)

        for r in pending_sends:
            r.wait_send()

        @functools.partial(pl.run_scoped, sem=pltpu.SemaphoreType.REGULAR)
        def _(sem):
            pl.semaphore_signal(
                sem, inc=1, device_id=(1 - ix, iy, iz),
                device_id_type=pl.DeviceIdType.MESH,
            )
            for dim in ("y", "z"):
                for _, _, dev in line_peers(dim):
                    pl.semaphore_signal(
                        sem, inc=1, device_id=dev,
                        device_id_type=pl.DeviceIdType.MESH,
                    )
            pl.semaphore_wait(sem, 7)

    return pl.pallas_call(
        body,
        out_shape=jax.ShapeDtypeStruct((M, N), jnp.float32),
        in_specs=[pl.BlockSpec(memory_space=pltpu.VMEM)],
        out_specs=pl.BlockSpec(memory_space=pltpu.VMEM),
        scratch_shapes=[
            pltpu.VMEM((M, N), jnp.bfloat16),
            pltpu.VMEM((M, N), jnp.bfloat16),
            pltpu.SemaphoreType.DMA((N_FLOWS, N_SEMS)),
            pltpu.SemaphoreType.DMA((N_FLOWS, N_SEMS)),
        ],
        compiler_params=pltpu.CompilerParams(collective_id=0),
    )(x)


# device time: 54391 ns/iter; 1.0227x vs baseline; 1.0227x over previous
import functools

import jax
import jax.numpy as jnp
from jax import lax
from jax.experimental import pallas as pl
from jax.experimental.pallas import tpu as pltpu

M, N = 1024, 1024
N_STAGES = 5
N_STEPS = 2 * N_STAGES
HALF_M = M // 2
HALVES = [HALF_M >> (k + 1) for k in range(N_STAGES)]
ORDER_A = ["x", "y1", "z1", "y2", "z2"]
ORDER_B = ["y1", "z1", "y2", "z2", "x"]
RSTART_A = [0, 256, 384, 448, 480]
RSTART_B = [512, 768, 896, 960, 992]


def kernel(x):
    def body(x_ref, out_ref, accum, recv, sa_send, sa_recv, sb_send, sb_recv):
        ix = lax.axis_index("x")
        iy = lax.axis_index("y")
        iz = lax.axis_index("z")
        stage_defs = {
            "x": ((1 - ix, iy, iz), ix),
            "y1": ((ix, iy ^ 1, iz), iy & 1),
            "z1": ((ix, iy, iz ^ 1), iz & 1),
            "y2": ((ix, iy ^ 2, iz), iy >> 1),
            "z2": ((ix, iy, iz ^ 2), iz >> 1),
        }
        halves = {
            "a": (ORDER_A, RSTART_A, sa_send, sa_recv),
            "b": (ORDER_B, RSTART_B, sb_send, sb_recv),
        }

        accum[...] = x_ref[0, 0, 0].astype(jnp.bfloat16)

        barrier_sem = pltpu.get_barrier_semaphore()
        for name in ORDER_A:
            pl.semaphore_signal(
                barrier_sem, inc=1, device_id=stage_defs[name][0],
                device_id_type=pl.DeviceIdType.MESH,
            )
        pl.semaphore_wait(barrier_sem, N_STAGES)

        def dsr(off, size):
            return pl.ds(pl.multiple_of(off, 16), size)

        pending_sends = []

        def start(h, t, off):
            order, rstart, send_sems, recv_sems = halves[h]
            if t < N_STAGES:
                k = t
                p, bk = stage_defs[order[k]]
                size = HALVES[k]
                send_off = off + (1 - bk) * size
                src = accum.at[dsr(send_off, size), :]
                dst = recv.at[dsr(rstart[k], size), :]
            else:
                k = N_STEPS - 1 - t
                p, _ = stage_defs[order[k]]
                size = HALVES[k]
                src = accum.at[dsr(off, size), :]
                dst = accum.at[dsr(off, size), :]
            rdma = pltpu.make_async_remote_copy(
                src_ref=src, dst_ref=dst,
                send_sem=send_sems.at[t], recv_sem=recv_sems.at[t],
                device_id=p, device_id_type=pl.DeviceIdType.MESH,
            )
            rdma.start()
            pending_sends.append(rdma)
            return rdma

        def process(h, t, off):
            order, rstart, _, _ = halves[h]
            if t < N_STAGES:
                k = t
                _, bk = stage_defs[order[k]]
                size = HALVES[k]
                keep_off = off + bk * size
                accum[dsr(keep_off, size), :] += recv[dsr(rstart[k], size), :]
                return keep_off
            k = N_STEPS - 1 - t
            _, bk = stage_defs[order[k]]
            return off - bk * HALVES[k]

        off_a = jnp.int32(0)
        off_b = jnp.int32(HALF_M)
        r_a = start("a", 0, off_a)
        r_b = start("b", 0, off_b)
        for t in range(N_STEPS):
            r_a.wait_recv()
            off_a = process("a", t, off_a)
            if t + 1 < N_STEPS:
                r_a = start("a", t + 1, off_a)
            r_b.wait_recv()
            off_b = process("b", t, off_b)
            if t + 1 < N_STEPS:
                r_b = start("b", t + 1, off_b)

        out_ref[...] = accum[...].astype(jnp.float32)

        for r in pending_sends:
            r.wait_send()

        @functools.partial(pl.run_scoped, sem=pltpu.SemaphoreType.REGULAR)
        def _(sem):
            for name in ORDER_A:
                pl.semaphore_signal(
                    sem, inc=1, device_id=stage_defs[name][0],
                    device_id_type=pl.DeviceIdType.MESH,
                )
            pl.semaphore_wait(sem, N_STAGES)

    return pl.pallas_call(
        body,
        out_shape=jax.ShapeDtypeStruct((M, N), jnp.float32),
        in_specs=[pl.BlockSpec(memory_space=pltpu.VMEM)],
        out_specs=pl.BlockSpec(memory_space=pltpu.VMEM),
        scratch_shapes=[
            pltpu.VMEM((M, N), jnp.bfloat16),
            pltpu.VMEM((M, N), jnp.bfloat16),
            pltpu.SemaphoreType.DMA((N_STEPS,)),
            pltpu.SemaphoreType.DMA((N_STEPS,)),
            pltpu.SemaphoreType.DMA((N_STEPS,)),
            pltpu.SemaphoreType.DMA((N_STEPS,)),
        ],
        compiler_params=pltpu.CompilerParams(collective_id=0),
    )(x)


# device time: 51903 ns/iter; 1.0717x vs baseline; 1.0479x over previous
import functools

import jax
import jax.numpy as jnp
from jax import lax
from jax.experimental import pallas as pl
from jax.experimental.pallas import tpu as pltpu

M, N = 1024, 1024
N_STAGES = 5
N_STEPS = 2 * N_STAGES
HALF_M = M // 2
COL_W = N // 4
HALVES = [HALF_M >> (k + 1) for k in range(N_STAGES)]
ORDER_A = ["x", "y1", "z1", "y2", "z2"]
ORDER_B = ["y1", "z1", "y2", "z2", "x"]
RSTART_A = [0, 256, 384, 448, 480]
RSTART_B = [512, 768, 896, 960, 992]
FLOWS = [
    (ORDER_A, RSTART_A, 0, 0 * COL_W, 0),
    (ORDER_B, RSTART_B, HALF_M, 0 * COL_W, 0),
    (ORDER_A, RSTART_A, 0, 1 * COL_W, 1),
    (ORDER_B, RSTART_B, HALF_M, 1 * COL_W, 1),
    (ORDER_A, RSTART_A, 0, 2 * COL_W, 2),
    (ORDER_B, RSTART_B, HALF_M, 2 * COL_W, 2),
    (ORDER_A, RSTART_A, 0, 3 * COL_W, 3),
    (ORDER_B, RSTART_B, HALF_M, 3 * COL_W, 3),
]
N_FLOWS = len(FLOWS)
MAX_LAG = max(f[4] for f in FLOWS)


def kernel(x):
    def body(x_ref, out_ref, accum, recv, send_sems, recv_sems):
        ix = lax.axis_index("x")
        iy = lax.axis_index("y")
        iz = lax.axis_index("z")
        stage_defs = {
            "x": ((1 - ix, iy, iz), ix),
            "y1": ((ix, iy ^ 1, iz), iy & 1),
            "z1": ((ix, iy, iz ^ 1), iz & 1),
            "y2": ((ix, iy ^ 2, iz), iy >> 1),
            "z2": ((ix, iy, iz ^ 2), iz >> 1),
        }

        accum[...] = x_ref[0, 0, 0].astype(jnp.bfloat16)

        barrier_sem = pltpu.get_barrier_semaphore()
        for name in ORDER_A:
            pl.semaphore_signal(
                barrier_sem, inc=1, device_id=stage_defs[name][0],
                device_id_type=pl.DeviceIdType.MESH,
            )
        pl.semaphore_wait(barrier_sem, N_STAGES)

        def dsr(off, size):
            return pl.ds(pl.multiple_of(off, 16), size)

        pending_sends = []

        def start(fi, t, off):
            order, rstart, _, c0, _ = FLOWS[fi]
            cols = pl.ds(c0, COL_W)
            if t < N_STAGES:
                k = t
                p, bk = stage_defs[order[k]]
                size = HALVES[k]
                send_off = off + (1 - bk) * size
                src = accum.at[dsr(send_off, size), cols]
                dst = recv.at[dsr(rstart[k], size), cols]
            else:
                k = N_STEPS - 1 - t
                p, _ = stage_defs[order[k]]
                size = HALVES[k]
                src = accum.at[dsr(off, size), cols]
                dst = accum.at[dsr(off, size), cols]
            rdma = pltpu.make_async_remote_copy(
                src_ref=src, dst_ref=dst,
                send_sem=send_sems.at[fi, t], recv_sem=recv_sems.at[fi, t],
                device_id=p, device_id_type=pl.DeviceIdType.MESH,
            )
            rdma.start()
            pending_sends.append(rdma)
            return rdma

        def process(fi, t, off):
            order, rstart, _, c0, _ = FLOWS[fi]
            cols = pl.ds(c0, COL_W)
            if t < N_STAGES:
                k = t
                _, bk = stage_defs[order[k]]
                size = HALVES[k]
                keep_off = off + bk * size
                accum[dsr(keep_off, size), cols] += (
                    recv[dsr(rstart[k], size), cols]
                )
                if k == N_STAGES - 1:
                    return keep_off, keep_off, size
                return keep_off, None, 0
            k = N_STEPS - 1 - t
            _, bk = stage_defs[order[k]]
            size = HALVES[k]
            sib_off = off + (1 - 2 * bk) * size
            return off - bk * size, sib_off, size

        def cast(fi, cast_off, cast_size):
            _, _, _, c0, _ = FLOWS[fi]
            cols = pl.ds(c0, COL_W)
            out_ref[dsr(cast_off, cast_size), cols] = (
                accum[dsr(cast_off, cast_size), cols]
            ).astype(jnp.float32)

        offs = [jnp.int32(FLOWS[fi][2]) for fi in range(N_FLOWS)]
        rdmas = [None] * N_FLOWS
        for fi in range(N_FLOWS):
            if FLOWS[fi][4] == 0:
                rdmas[fi] = start(fi, 0, offs[fi])
        for tau in range(N_STEPS + MAX_LAG):
            for fi in range(N_FLOWS):
                t = tau - FLOWS[fi][4]
                if t < 0 or t >= N_STEPS:
                    continue
                rdmas[fi].wait_recv()
                offs[fi], cast_off, cast_size = process(fi, t, offs[fi])
                if t + 1 < N_STEPS:
                    rdmas[fi] = start(fi, t + 1, offs[fi])
                if cast_size:
                    cast(fi, cast_off, cast_size)
            for fi in range(N_FLOWS):
                if FLOWS[fi][4] == tau + 1:
                    rdmas[fi] = start(fi, 0, offs[fi])

        for r in pending_sends:
            r.wait_send()

        @functools.partial(pl.run_scoped, sem=pltpu.SemaphoreType.REGULAR)
        def _(sem):
            for name in ORDER_A:
                pl.semaphore_signal(
                    sem, inc=1, device_id=stage_defs[name][0],
                    device_id_type=pl.DeviceIdType.MESH,
                )
            pl.semaphore_wait(sem, N_STAGES)

    return pl.pallas_call(
        body,
        out_shape=jax.ShapeDtypeStruct((M, N), jnp.float32),
        in_specs=[pl.BlockSpec(memory_space=pltpu.VMEM)],
        out_specs=pl.BlockSpec(memory_space=pltpu.VMEM),
        scratch_shapes=[
            pltpu.VMEM((M, N), jnp.bfloat16),
            pltpu.VMEM((M, N), jnp.bfloat16),
            pltpu.SemaphoreType.DMA((N_FLOWS, N_STEPS)),
            pltpu.SemaphoreType.DMA((N_FLOWS, N_STEPS)),
        ],
        compiler_params=pltpu.CompilerParams(collective_id=0),
    )(x)


# device time: 51353 ns/iter; 1.0832x vs baseline; 1.0107x over previous
import functools

import jax
import jax.numpy as jnp
from jax import lax
from jax.experimental import pallas as pl
from jax.experimental.pallas import tpu as pltpu

M, N = 1024, 1024
N_STAGES = 5
N_STEPS = 2 * N_STAGES
HALF_M = M // 2
COL_W = N // 2
HALVES = [HALF_M >> (k + 1) for k in range(N_STAGES)]
ORDER_A = ["x", "y1", "z1", "y2", "z2"]
ORDER_B = ["y1", "z1", "y2", "z2", "x"]
RSTART_A = [0, 256, 384, 448, 480]
RSTART_B = [512, 768, 896, 960, 992]
FLOWS = [
    (ORDER_A, RSTART_A, 0, 0, 0),
    (ORDER_B, RSTART_B, HALF_M, 0, 0),
    (ORDER_A, RSTART_A, 0, COL_W, 1),
    (ORDER_B, RSTART_B, HALF_M, COL_W, 1),
]
N_FLOWS = len(FLOWS)
MAX_LAG = max(f[4] for f in FLOWS)


def kernel(x):
    def body(x_ref, out_ref, accum, recv, send_sems, recv_sems):
        ix = lax.axis_index("x")
        iy = lax.axis_index("y")
        iz = lax.axis_index("z")
        stage_defs = {
            "x": ((1 - ix, iy, iz), ix),
            "y1": ((ix, iy ^ 1, iz), iy & 1),
            "z1": ((ix, iy, iz ^ 1), iz & 1),
            "y2": ((ix, iy ^ 2, iz), iy >> 1),
            "z2": ((ix, iy, iz ^ 2), iz >> 1),
        }

        accum[...] = x_ref[0, 0, 0].astype(jnp.bfloat16)

        barrier_sem = pltpu.get_barrier_semaphore()
        for name in ORDER_A:
            pl.semaphore_signal(
                barrier_sem, inc=1, device_id=stage_defs[name][0],
                device_id_type=pl.DeviceIdType.MESH,
            )
        pl.semaphore_wait(barrier_sem, N_STAGES)

        def dsr(off, size):
            return pl.ds(pl.multiple_of(off, 16), size)

        offs = []
        for fi in range(N_FLOWS):
            order, _, row_base, _, _ = FLOWS[fi]
            o = [jnp.int32(row_base)]
            for k in range(N_STAGES):
                o.append(o[-1] + stage_defs[order[k]][1] * HALVES[k])
            offs.append(o)

        descs = []
        for fi in range(N_FLOWS):
            order, rstart, _, c0, _ = FLOWS[fi]
            cols = pl.ds(c0, COL_W)
            d = []
            for t in range(N_STEPS):
                if t < N_STAGES:
                    k = t
                    p, bk = stage_defs[order[k]]
                    size = HALVES[k]
                    send_off = offs[fi][k] + (1 - bk) * size
                    src = accum.at[dsr(send_off, size), cols]
                    dst = recv.at[dsr(rstart[k], size), cols]
                else:
                    k = N_STEPS - 1 - t
                    p, _ = stage_defs[order[k]]
                    size = HALVES[k]
                    src = accum.at[dsr(offs[fi][k + 1], size), cols]
                    dst = accum.at[dsr(offs[fi][k + 1], size), cols]
                d.append(
                    pltpu.make_async_remote_copy(
                        src_ref=src, dst_ref=dst,
                        send_sem=send_sems.at[fi, t],
                        recv_sem=recv_sems.at[fi, t],
                        device_id=p, device_id_type=pl.DeviceIdType.MESH,
                    )
                )
            descs.append(d)

        def process(fi, t):
            order, rstart, _, c0, _ = FLOWS[fi]
            cols = pl.ds(c0, COL_W)
            if t < N_STAGES:
                k = t
                size = HALVES[k]
                keep_off = offs[fi][k + 1]
                accum[dsr(keep_off, size), cols] += (
                    recv[dsr(rstart[k], size), cols]
                )
                if k == N_STAGES - 1:
                    return keep_off, size
                return None, 0
            k = N_STEPS - 1 - t
            _, bk = stage_defs[order[k]]
            size = HALVES[k]
            return offs[fi][k] + (1 - bk) * size, size

        def cast(fi, cast_off, cast_size):
            _, _, _, c0, _ = FLOWS[fi]
            cols = pl.ds(c0, COL_W)
            out_ref[dsr(cast_off, cast_size), cols] = (
                accum[dsr(cast_off, cast_size), cols]
            ).astype(jnp.float32)

        for fi in range(N_FLOWS):
            if FLOWS[fi][4] == 0:
                descs[fi][0].start()
        for tau in range(N_STEPS + MAX_LAG):
            for fi in range(N_FLOWS):
                t = tau - FLOWS[fi][4]
                if t < 0 or t >= N_STEPS:
                    continue
                descs[fi][t].wait_recv()
                cast_off, cast_size = process(fi, t)
                if t + 1 < N_STEPS:
                    descs[fi][t + 1].start()
                if cast_size:
                    cast(fi, cast_off, cast_size)
            for fi in range(N_FLOWS):
                if FLOWS[fi][4] == tau + 1:
                    descs[fi][0].start()

        for fi in range(N_FLOWS):
            for t in range(N_STEPS):
                descs[fi][t].wait_send()

        @functools.partial(pl.run_scoped, sem=pltpu.SemaphoreType.REGULAR)
        def _(sem):
            for name in ORDER_A:
                pl.semaphore_signal(
                    sem, inc=1, device_id=stage_defs[name][0],
                    device_id_type=pl.DeviceIdType.MESH,
                )
            pl.semaphore_wait(sem, N_STAGES)

    return pl.pallas_call(
        body,
        out_shape=jax.ShapeDtypeStruct((M, N), jnp.float32),
        in_specs=[pl.BlockSpec(memory_space=pltpu.VMEM)],
        out_specs=pl.BlockSpec(memory_space=pltpu.VMEM),
        scratch_shapes=[
            pltpu.VMEM((M, N), jnp.bfloat16),
            pltpu.VMEM((M, N), jnp.bfloat16),
            pltpu.SemaphoreType.DMA((N_FLOWS, N_STEPS)),
            pltpu.SemaphoreType.DMA((N_FLOWS, N_STEPS)),
        ],
        compiler_params=pltpu.CompilerParams(collective_id=0),
    )(x)
